# baseline (speedup 1.0000x reference)
import functools

import jax
import jax.numpy as jnp
from jax import lax
from jax.experimental import pallas as pl
from jax.experimental.pallas import tpu as pltpu
from jax.experimental.pallas import tpu_sc as plsc

VOCAB = 1000
VPAD = 1024
T = 95
TP = 96
D = 64
BATCH = 16384
PPAD = 6144

NC, NS, L = 2, 16, 16
NW = NC * NS
BCOLS = BATCH // NW
CB = 256
NCHUNK = BCOLS // CB
NB16 = CB // L


def _sc_embed(ids_tp, tok_flat, pos_flat):
    mesh = plsc.VectorSubcoreMesh(core_axis_name="c", subcore_axis_name="s")

    scratch = [
        pltpu.VMEM((D * VPAD,), jnp.float32),
        pltpu.VMEM((PPAD,), jnp.float32),
        pltpu.VMEM((TP, CB), jnp.int32),
        pltpu.VMEM((D, CB), jnp.float32),
        pltpu.VMEM((D, CB), jnp.float32),
        pltpu.SemaphoreType.DMA,
        pltpu.SemaphoreType.DMA,
        pltpu.SemaphoreType.DMA,
    ]

    @functools.partial(
        pl.kernel,
        mesh=mesh,
        compiler_params=pltpu.CompilerParams(needs_layout_passes=False),
        out_type=jax.ShapeDtypeStruct((T, D, BATCH), jnp.float32),
        scratch_types=scratch,
    )
    def k(ids_hbm, tok_hbm, pos_hbm, out_hbm, tokf_v, posf_v, ids_v,
          ob0, ob1, tsem, osem0, osem1):
        obuf = (ob0, ob1)
        osem = (osem0, osem1)
        wid = lax.axis_index("s") * NC + lax.axis_index("c")

        stage = [
            pltpu.make_async_copy(tok_hbm, tokf_v, tsem),
            pltpu.make_async_copy(pos_hbm, posf_v, tsem),
        ]
        for cp in stage:
            cp.start()
        for cp in stage:
            cp.wait()

        def o_copy(t, b0, p):
            return pltpu.make_async_copy(
                obuf[p], out_hbm.at[t, :, pl.ds(b0, CB)], osem[p])

        def compute(t, p):
            idvs = [ids_v[t, pl.ds(i * L, L)] for i in range(NB16)]
            ob = obuf[p]

            @plsc.parallel_loop(0, D, step=2, unroll=1)
            def dblock(d):
                pvec = posf_v[pl.ds(t * D + d, L)]
                for j in range(2):
                    pb = jnp.full((L,), pvec[j], jnp.float32)
                    dm = jnp.full((L,), j * VPAD, jnp.int32) + d * VPAD
                    for i in range(NB16):
                        g = plsc.load_gather(tokf_v, [idvs[i] + dm])
                        ob[d + j, pl.ds(i * L, L)] = g + pb

        def chunk_body(chunk, carry):
            b0 = wid * BCOLS + chunk * CB
            pltpu.sync_copy(ids_hbm.at[:, pl.ds(b0, CB)], ids_v)

            compute(0, 0)
            o_copy(0, b0, 0).start()
            compute(1, 1)
            o_copy(1, b0, 1).start()

            def body(i, carry2):
                t = 2 + 2 * i
                o_copy(t - 2, b0, 0).wait()
                compute(t, 0)
                o_copy(t, b0, 0).start()
                o_copy(t - 1, b0, 1).wait()
                compute(t + 1, 1)
                o_copy(t + 1, b0, 1).start()
                return carry2

            lax.fori_loop(0, (T - 3) // 2, body, 0)

            o_copy(T - 3, b0, 0).wait()
            compute(T - 1, 0)
            o_copy(T - 1, b0, 0).start()
            o_copy(T - 2, b0, 1).wait()
            o_copy(T - 1, b0, 0).wait()
            return carry

        lax.fori_loop(0, NCHUNK, chunk_body, 0)

    return k(ids_tp, tok_flat, pos_flat)


def kernel(position_token_ids, token_embedding, position_embedding):
    ids_tp = jnp.pad(position_token_ids.astype(jnp.int32).T,
                     ((0, TP - T), (0, 0)))
    tok_flat = jnp.pad(token_embedding.T,
                       ((0, 0), (0, VPAD - VOCAB))).reshape(-1)
    pos_flat = jnp.pad(position_embedding.reshape(-1),
                       (0, PPAD - T * D))
    out_tdb = _sc_embed(ids_tp, tok_flat, pos_flat)
    return out_tdb.transpose(2, 0, 1)


# speedup vs baseline: 1.0667x; 1.0667x over previous
import functools

import jax
import jax.numpy as jnp
from jax import lax
from jax.experimental import pallas as pl
from jax.experimental.pallas import tpu as pltpu
from jax.experimental.pallas import tpu_sc as plsc

VOCAB = 1000
VPAD = 1024
T = 95
TP = 96
D = 64
BATCH = 16384
PPAD = 6144

NC, NS, L = 2, 16, 16
NW = NC * NS
BCOLS = BATCH // NW
CB = 256
NCHUNK = BCOLS // CB
NB16 = CB // L


def _sc_embed(ids_tp, tok_flat, pos_flat):
    mesh = plsc.VectorSubcoreMesh(core_axis_name="c", subcore_axis_name="s")

    scratch = [
        pltpu.VMEM((D * VPAD,), jnp.float32),
        pltpu.VMEM((PPAD,), jnp.float32),
        pltpu.VMEM((TP, CB), jnp.int32),
        pltpu.VMEM((D, CB), jnp.float32),
        pltpu.VMEM((D, CB), jnp.float32),
        pltpu.SemaphoreType.DMA,
        pltpu.SemaphoreType.DMA,
        pltpu.SemaphoreType.DMA,
    ]

    @functools.partial(
        pl.kernel,
        mesh=mesh,
        compiler_params=pltpu.CompilerParams(needs_layout_passes=False),
        out_type=jax.ShapeDtypeStruct((T, D, BATCH), jnp.float32),
        scratch_types=scratch,
    )
    def k(ids_hbm, tok_hbm, pos_hbm, out_hbm, tokf_v, posf_v, ids_v,
          ob0, ob1, tsem, osem0, osem1):
        obuf = (ob0, ob1)
        osem = (osem0, osem1)
        wid = lax.axis_index("s") * NC + lax.axis_index("c")

        stage = [
            pltpu.make_async_copy(tok_hbm, tokf_v, tsem),
            pltpu.make_async_copy(pos_hbm, posf_v, tsem),
        ]
        for cp in stage:
            cp.start()
        for cp in stage:
            cp.wait()

        def o_copy(t, b0, p):
            return pltpu.make_async_copy(
                obuf[p], out_hbm.at[t, :, pl.ds(b0, CB)], osem[p])

        def compute(t, p):
            idvs = [ids_v[t, pl.ds(i * L, L)] for i in range(NB16)]
            ob = obuf[p]

            @plsc.parallel_loop(0, D, step=2, unroll=2)
            def dblock(d):
                pvec = posf_v[pl.ds(t * D + d, L)]
                for j in range(2):
                    pb = jnp.full((L,), pvec[j], jnp.float32)
                    dm = jnp.full((L,), j * VPAD, jnp.int32) + d * VPAD
                    for i in range(NB16):
                        g = plsc.load_gather(tokf_v, [idvs[i] + dm])
                        ob[d + j, pl.ds(i * L, L)] = g + pb

        def chunk_body(chunk, carry):
            b0 = wid * BCOLS + chunk * CB
            pltpu.sync_copy(ids_hbm.at[:, pl.ds(b0, CB)], ids_v)

            compute(0, 0)
            o_copy(0, b0, 0).start()
            compute(1, 1)
            o_copy(1, b0, 1).start()

            def body(i, carry2):
                t = 2 + 2 * i
                o_copy(t - 2, b0, 0).wait()
                compute(t, 0)
                o_copy(t, b0, 0).start()
                o_copy(t - 1, b0, 1).wait()
                compute(t + 1, 1)
                o_copy(t + 1, b0, 1).start()
                return carry2

            lax.fori_loop(0, (T - 3) // 2, body, 0)

            o_copy(T - 3, b0, 0).wait()
            compute(T - 1, 0)
            o_copy(T - 1, b0, 0).start()
            o_copy(T - 2, b0, 1).wait()
            o_copy(T - 1, b0, 0).wait()
            return carry

        lax.fori_loop(0, NCHUNK, chunk_body, 0)

    return k(ids_tp, tok_flat, pos_flat)


def kernel(position_token_ids, token_embedding, position_embedding):
    ids_tp = jnp.pad(position_token_ids.astype(jnp.int32).T,
                     ((0, TP - T), (0, 0)))
    tok_flat = jnp.pad(token_embedding.T,
                       ((0, 0), (0, VPAD - VOCAB))).reshape(-1)
    pos_flat = jnp.pad(position_embedding.reshape(-1),
                       (0, PPAD - T * D))
    out_tdb = _sc_embed(ids_tp, tok_flat, pos_flat)
    return out_tdb.transpose(2, 0, 1)
